# Initial kernel scaffold; baseline (speedup 1.0000x reference)
import jax
import jax.numpy as jnp
from jax.experimental import pallas as pl


def kernel(z, edge_index, batch):
    raise NotImplementedError("write your pallas kernel here")


# speedup vs baseline: 5.8641x; 5.8641x over previous
import functools

import jax
import jax.numpy as jnp
from jax import lax
from jax.experimental import pallas as pl
from jax.experimental.pallas import tpu as pltpu
from jax.experimental.pallas import tpu_sc as plsc

_EPS = 1e-05
_NUM_GRAPHS = 16
_N, _D, _E = 10000, 128, 320000
_NC, _NS, _L = 2, 16, 16
_NW = _NC * _NS
_EW = _E // _NW
_CHUNK = 80
_NCHUNK = _EW // _CHUNK


def _sc_mesh():
    return plsc.VectorSubcoreMesh(core_axis_name="c", subcore_axis_name="s",
                                  num_cores=_NC, num_subcores=_NS)


@functools.partial(
    pl.kernel,
    out_type=[jax.ShapeDtypeStruct((_E,), jnp.float32),
              jax.ShapeDtypeStruct((_E,), jnp.float32)],
    mesh=_sc_mesh(),
    scratch_types=[
        pltpu.VMEM((_N,), jnp.int32),
        pltpu.VMEM((_NUM_GRAPHS,), jnp.int32),
        pltpu.VMEM((_NUM_GRAPHS,), jnp.int32),
        pltpu.VMEM((_CHUNK,), jnp.int32),
        pltpu.VMEM((_CHUNK,), jnp.int32),
        pltpu.VMEM((_CHUNK,), jnp.int32),
        pltpu.VMEM((_CHUNK,), jnp.float32),
        pltpu.VMEM((_CHUNK, _D), jnp.float32),
        pltpu.VMEM((_CHUNK, _D), jnp.float32),
        pltpu.VMEM((_CHUNK, _D), jnp.float32),
        pltpu.VMEM((_CHUNK,), jnp.float32),
        pltpu.VMEM((_CHUNK,), jnp.float32),
        pltpu.SemaphoreType.DMA,
        pltpu.SemaphoreType.DMA,
        pltpu.SemaphoreType.DMA,
    ],
    compiler_params=pltpu.CompilerParams(needs_layout_passes=False),
)
def _sc_dots(z_hbm, src_hbm, dst_hbm, u_hbm, batch_hbm, cnt_hbm, start_hbm,
             pos_hbm, negv_hbm,
             batch_v, cnt_v, start_v, srci_v, dsti_v, negi_v, u_v,
             srow, drow, nrow, posb, negb, sem_s, sem_d, sem_n):
    wid = lax.axis_index("s") * jnp.int32(_NC) + lax.axis_index("c")
    base = wid * jnp.int32(_EW)
    pltpu.sync_copy(batch_hbm, batch_v)
    pltpu.sync_copy(cnt_hbm, cnt_v)
    pltpu.sync_copy(start_hbm, start_v)
    lane = lax.iota(jnp.int32, _L)
    lane0 = lane == 0

    def chunk_body(j, carry):
        off = base + j * jnp.int32(_CHUNK)
        pltpu.sync_copy(src_hbm.at[pl.ds(off, _CHUNK)], srci_v)
        pltpu.sync_copy(dst_hbm.at[pl.ds(off, _CHUNK)], dsti_v)
        pltpu.sync_copy(u_hbm.at[pl.ds(off, _CHUNK)], u_v)
        cp_s = pltpu.async_copy(z_hbm.at[srci_v], srow, sem_s)
        cp_d = pltpu.async_copy(z_hbm.at[dsti_v], drow, sem_d)
        for t in range(_CHUNK // _L):
            sl = pl.ds(t * _L, _L)
            g = plsc.load_gather(batch_v, [srci_v[sl]])
            cnt = plsc.load_gather(cnt_v, [g])
            st = plsc.load_gather(start_v, [g])
            r = (u_v[sl] * cnt.astype(jnp.float32)).astype(jnp.int32)
            negi_v[sl] = st + jnp.minimum(r, cnt - 1)
        cp_n = pltpu.async_copy(z_hbm.at[negi_v], nrow, sem_n)
        cp_s.wait()
        cp_d.wait()
        cp_n.wait()

        def edge_body(e, c):
            accp = jnp.zeros((_L,), jnp.float32)
            accn = jnp.zeros((_L,), jnp.float32)
            for k in range(_D // _L):
                ks = pl.ds(k * _L, _L)
                s = srow[e, ks]
                accp = accp + s * drow[e, ks]
                accn = accn + s * nrow[e, ks]
            pv = jnp.sum(accp)
            nv = jnp.sum(accn)
            ev = jnp.full((_L,), e, jnp.int32)
            plsc.store_scatter(posb, [ev], jnp.full((_L,), pv, jnp.float32),
                               mask=lane0)
            plsc.store_scatter(negb, [ev], jnp.full((_L,), nv, jnp.float32),
                               mask=lane0)
            return c

        lax.fori_loop(jnp.int32(0), jnp.int32(_CHUNK), edge_body, jnp.int32(0))
        pltpu.sync_copy(posb, pos_hbm.at[pl.ds(off, _CHUNK)])
        pltpu.sync_copy(negb, negv_hbm.at[pl.ds(off, _CHUNK)])
        return carry

    lax.fori_loop(jnp.int32(0), jnp.int32(_NCHUNK), chunk_body, jnp.int32(0))


def _tc_reduce_body(pos_ref, neg_ref, out_ref):
    p = pos_ref[...]
    q = neg_ref[...]
    pos_term = -jnp.log(jax.nn.sigmoid(p) + _EPS)
    neg_term = -jnp.log(1.0 - jax.nn.sigmoid(q) + _EPS)
    out_ref[0] = (jnp.sum(pos_term) + jnp.sum(neg_term)) / _E


_tc_reduce = pl.pallas_call(
    _tc_reduce_body,
    out_shape=jax.ShapeDtypeStruct((1,), jnp.float32),
    out_specs=pl.BlockSpec(memory_space=pltpu.SMEM),
)


def kernel(z, edge_index, batch):
    z32 = z.astype(jnp.float32)
    src = edge_index[0].astype(jnp.int32)
    dst = edge_index[1].astype(jnp.int32)
    b32 = batch.astype(jnp.int32)
    counts = jnp.bincount(b32, length=_NUM_GRAPHS)
    starts = (jnp.cumsum(counts) - counts).astype(jnp.int32)
    cnt_tab = jnp.maximum(counts, 1).astype(jnp.int32)
    u = jax.random.uniform(jax.random.key(42), (_E,), dtype=jnp.float32)
    pos_v, neg_v = _sc_dots(z32, src, dst, u, b32, cnt_tab, starts)
    loss = _tc_reduce(pos_v.reshape(_E // _D, _D), neg_v.reshape(_E // _D, _D))
    return loss[0]


# speedup vs baseline: 10.1974x; 1.7390x over previous
import functools

import jax
import jax.numpy as jnp
from jax import lax
from jax.experimental import pallas as pl
from jax.experimental.pallas import tpu as pltpu
from jax.experimental.pallas import tpu_sc as plsc

_EPS = 1e-05
_NUM_GRAPHS = 16
_N, _D, _E = 10000, 128, 320000
_NC, _NS, _L = 2, 16, 16
_NW = _NC * _NS
_EW = _E // _NW
_CHUNK = 80
_NCHUNK = _EW // _CHUNK


def _sc_mesh():
    return plsc.VectorSubcoreMesh(core_axis_name="c", subcore_axis_name="s",
                                  num_cores=_NC, num_subcores=_NS)


@functools.partial(
    pl.kernel,
    out_type=[jax.ShapeDtypeStruct((_E,), jnp.float32),
              jax.ShapeDtypeStruct((_E,), jnp.float32)],
    mesh=_sc_mesh(),
    scratch_types=[
        pltpu.VMEM((_N,), jnp.int32),
        pltpu.VMEM((_NUM_GRAPHS,), jnp.int32),
        pltpu.VMEM((_NUM_GRAPHS,), jnp.int32),
        pltpu.VMEM((_CHUNK,), jnp.int32),
        pltpu.VMEM((_CHUNK,), jnp.int32),
        pltpu.VMEM((_CHUNK,), jnp.int32),
        pltpu.VMEM((_CHUNK,), jnp.float32),
        pltpu.VMEM((_CHUNK, _D), jnp.float32),
        pltpu.VMEM((_CHUNK, _D), jnp.float32),
        pltpu.VMEM((_CHUNK, _D), jnp.float32),
        pltpu.VMEM((_CHUNK,), jnp.float32),
        pltpu.VMEM((_CHUNK,), jnp.float32),
        pltpu.SemaphoreType.DMA,
        pltpu.SemaphoreType.DMA,
        pltpu.SemaphoreType.DMA,
    ],
    compiler_params=pltpu.CompilerParams(needs_layout_passes=False),
)
def _sc_dots(z_hbm, src_hbm, dst_hbm, u_hbm, batch_hbm, cnt_hbm, start_hbm,
             pos_hbm, negv_hbm,
             batch_v, cnt_v, start_v, srci_v, dsti_v, negi_v, u_v,
             srow, drow, nrow, posb, negb, sem_s, sem_d, sem_n):
    wid = lax.axis_index("s") * jnp.int32(_NC) + lax.axis_index("c")
    base = wid * jnp.int32(_EW)
    pltpu.sync_copy(batch_hbm, batch_v)
    pltpu.sync_copy(cnt_hbm, cnt_v)
    pltpu.sync_copy(start_hbm, start_v)
    lane = lax.iota(jnp.int32, _L)
    lane0 = lane == 0

    def chunk_body(j, carry):
        off = base + j * jnp.int32(_CHUNK)
        pltpu.sync_copy(src_hbm.at[pl.ds(off, _CHUNK)], srci_v)
        pltpu.sync_copy(dst_hbm.at[pl.ds(off, _CHUNK)], dsti_v)
        pltpu.sync_copy(u_hbm.at[pl.ds(off, _CHUNK)], u_v)
        cp_s = pltpu.async_copy(z_hbm.at[srci_v], srow, sem_s)
        cp_d = pltpu.async_copy(z_hbm.at[dsti_v], drow, sem_d)
        for t in range(_CHUNK // _L):
            sl = pl.ds(t * _L, _L)
            g = plsc.load_gather(batch_v, [srci_v[sl]])
            cnt = plsc.load_gather(cnt_v, [g])
            st = plsc.load_gather(start_v, [g])
            r = (u_v[sl] * cnt.astype(jnp.float32)).astype(jnp.int32)
            negi_v[sl] = st + jnp.minimum(r, cnt - 1)
        cp_n = pltpu.async_copy(z_hbm.at[negi_v], nrow, sem_n)
        cp_s.wait()
        cp_d.wait()
        cp_n.wait()

        def edge_body(e, c):
            accp = jnp.zeros((_L,), jnp.float32)
            accn = jnp.zeros((_L,), jnp.float32)
            for k in range(_D // _L):
                ks = pl.ds(k * _L, _L)
                s = srow[e, ks]
                accp = accp + s * drow[e, ks]
                accn = accn + s * nrow[e, ks]
            pv = jnp.sum(accp)
            nv = jnp.sum(accn)
            ev = jnp.full((_L,), e, jnp.int32)
            plsc.store_scatter(posb, [ev], jnp.full((_L,), pv, jnp.float32),
                               mask=lane0)
            plsc.store_scatter(negb, [ev], jnp.full((_L,), nv, jnp.float32),
                               mask=lane0)
            return c

        lax.fori_loop(jnp.int32(0), jnp.int32(_CHUNK), edge_body, jnp.int32(0))
        pltpu.sync_copy(posb, pos_hbm.at[pl.ds(off, _CHUNK)])
        pltpu.sync_copy(negb, negv_hbm.at[pl.ds(off, _CHUNK)])
        return carry

    lax.fori_loop(jnp.int32(0), jnp.int32(_NCHUNK), chunk_body, jnp.int32(0))


def _tc_reduce_body(pos_ref, neg_ref, out_ref):
    p = pos_ref[...]
    q = neg_ref[...]
    pos_term = -jnp.log(jax.nn.sigmoid(p) + _EPS)
    neg_term = -jnp.log(1.0 - jax.nn.sigmoid(q) + _EPS)
    out_ref[0] = (jnp.sum(pos_term) + jnp.sum(neg_term)) / _E


_tc_reduce = pl.pallas_call(
    _tc_reduce_body,
    out_shape=jax.ShapeDtypeStruct((1,), jnp.float32),
    out_specs=pl.BlockSpec(memory_space=pltpu.SMEM),
)


def kernel(z, edge_index, batch):
    z32 = z.astype(jnp.float32)
    src = edge_index[0].astype(jnp.int32)
    dst = edge_index[1].astype(jnp.int32)
    b32 = batch.astype(jnp.int32)
    counts = jnp.sum(
        b32[None, :] == jnp.arange(_NUM_GRAPHS, dtype=jnp.int32)[:, None],
        axis=1, dtype=jnp.int32)
    starts = (jnp.cumsum(counts) - counts).astype(jnp.int32)
    cnt_tab = jnp.maximum(counts, 1).astype(jnp.int32)
    u = jax.random.uniform(jax.random.key(42), (_E,), dtype=jnp.float32)
    pos_v, neg_v = _sc_dots(z32, src, dst, u, b32, cnt_tab, starts)
    loss = _tc_reduce(pos_v.reshape(_E // _D, _D), neg_v.reshape(_E // _D, _D))
    return loss[0]


# speedup vs baseline: 21.8032x; 2.1381x over previous
import functools

import jax
import jax.numpy as jnp
from jax import lax
from jax.experimental import pallas as pl
from jax.experimental.pallas import tpu as pltpu
from jax.experimental.pallas import tpu_sc as plsc

_EPS = 1e-05
_NUM_GRAPHS = 16
_N, _D, _E = 10000, 128, 320000
_NC, _NS, _L = 2, 16, 16
_NW = _NC * _NS
_EW = _E // _NW
_CHUNK = 80
_NCHUNK = _EW // _CHUNK
_UNROLL = 4


def _sc_mesh():
    return plsc.VectorSubcoreMesh(core_axis_name="c", subcore_axis_name="s",
                                  num_cores=_NC, num_subcores=_NS)


@functools.partial(
    pl.kernel,
    out_type=[jax.ShapeDtypeStruct((_E,), jnp.float32),
              jax.ShapeDtypeStruct((_E,), jnp.float32)],
    mesh=_sc_mesh(),
    scratch_types=[
        pltpu.VMEM((_NUM_GRAPHS,), jnp.int32),
        pltpu.VMEM((_NUM_GRAPHS,), jnp.int32),
        pltpu.VMEM((_EW,), jnp.int32),
        pltpu.VMEM((_EW,), jnp.int32),
        pltpu.VMEM((_EW,), jnp.int32),
        pltpu.VMEM((_EW,), jnp.float32),
        pltpu.VMEM((_CHUNK, _D), jnp.float32),
        pltpu.VMEM((_CHUNK, _D), jnp.float32),
        pltpu.VMEM((_CHUNK, _D), jnp.float32),
        pltpu.VMEM((_CHUNK, _D), jnp.float32),
        pltpu.VMEM((_CHUNK, _D), jnp.float32),
        pltpu.VMEM((_CHUNK, _D), jnp.float32),
        pltpu.VMEM((_EW,), jnp.float32),
        pltpu.VMEM((_EW,), jnp.float32),
        pltpu.SemaphoreType.DMA,
        pltpu.SemaphoreType.DMA,
        pltpu.SemaphoreType.DMA,
    ],
    compiler_params=pltpu.CompilerParams(needs_layout_passes=False),
)
def _sc_dots(z_hbm, src_hbm, dst_hbm, u_hbm, cnt_hbm, start_hbm,
             pos_hbm, negv_hbm,
             cnt_v, start_v, srci_v, dsti_v, negi_v, u_v,
             srow0, drow0, nrow0, srow1, drow1, nrow1, posacc, negacc,
             sem_in, sem_g0, sem_g1):
    wid = lax.axis_index("s") * jnp.int32(_NC) + lax.axis_index("c")
    base = wid * jnp.int32(_EW)
    lane = lax.iota(jnp.int32, _L)
    lane0 = lane == 0

    pltpu.sync_copy(cnt_hbm, cnt_v)
    pltpu.sync_copy(start_hbm, start_v)
    cp_a = pltpu.async_copy(src_hbm.at[pl.ds(base, _EW)], srci_v, sem_in)
    cp_b = pltpu.async_copy(dst_hbm.at[pl.ds(base, _EW)], dsti_v, sem_in)
    cp_c = pltpu.async_copy(u_hbm.at[pl.ds(base, _EW)], u_v, sem_in)
    cp_a.wait()
    cp_b.wait()
    cp_c.wait()

    def neg_body(t, c):
        sl = pl.ds(t * jnp.int32(_L), _L)
        sv = srci_v[sl]
        g = jnp.zeros((_L,), jnp.int32)
        for bit in (8, 4, 2, 1):
            probe = g | jnp.int32(bit)
            vals = plsc.load_gather(start_v, [probe])
            g = jnp.where(vals <= sv, probe, g)
        cnt = plsc.load_gather(cnt_v, [g])
        st = plsc.load_gather(start_v, [g])
        r = (u_v[sl] * cnt.astype(jnp.float32)).astype(jnp.int32)
        negi_v[sl] = st + jnp.minimum(r, cnt - 1)
        return c

    lax.fori_loop(jnp.int32(0), jnp.int32(_EW // _L), neg_body, jnp.int32(0))

    bufs = ((srow0, drow0, nrow0, sem_g0), (srow1, drow1, nrow1, sem_g1))

    def gather_descs(j, b):
        sr, dr, nr, sem = bufs[b]
        sl = pl.ds(j * jnp.int32(_CHUNK), _CHUNK)
        return (pltpu.make_async_copy(z_hbm.at[srci_v.at[sl]], sr, sem),
                pltpu.make_async_copy(z_hbm.at[dsti_v.at[sl]], dr, sem),
                pltpu.make_async_copy(z_hbm.at[negi_v.at[sl]], nr, sem))

    def issue(j, b):
        for c in gather_descs(j, b):
            c.start()

    def drain(j, b):
        for c in gather_descs(j, b):
            c.wait()

    def compute(j, b):
        sr, dr, nr, _ = bufs[b]
        ebase = j * jnp.int32(_CHUNK)

        def edge_body(t, c):
            e0 = t * jnp.int32(_UNROLL)
            for i in range(_UNROLL):
                e = e0 + jnp.int32(i)
                accp = None
                accn = None
                for k in range(_D // _L):
                    ks = pl.ds(k * _L, _L)
                    s = sr[e, ks]
                    pp = s * dr[e, ks]
                    pn = s * nr[e, ks]
                    accp = pp if accp is None else accp + pp
                    accn = pn if accn is None else accn + pn
                pv = jnp.sum(accp)
                nv = jnp.sum(accn)
                ev = jnp.full((_L,), ebase + e, jnp.int32)
                plsc.store_scatter(posacc, [ev],
                                   jnp.full((_L,), pv, jnp.float32), mask=lane0)
                plsc.store_scatter(negacc, [ev],
                                   jnp.full((_L,), nv, jnp.float32), mask=lane0)
            return c

        lax.fori_loop(jnp.int32(0), jnp.int32(_CHUNK // _UNROLL), edge_body,
                      jnp.int32(0))

    issue(jnp.int32(0), 0)
    issue(jnp.int32(1), 1)

    def chunk_pair(t, c):
        jj = t * jnp.int32(2)
        drain(jj, 0)

        @pl.when(jj + 2 < _NCHUNK)
        def _():
            issue(jj + jnp.int32(2), 0)

        compute(jj, 0)
        drain(jj + jnp.int32(1), 1)

        @pl.when(jj + 3 < _NCHUNK)
        def _():
            issue(jj + jnp.int32(3), 1)

        compute(jj + jnp.int32(1), 1)
        return c

    lax.fori_loop(jnp.int32(0), jnp.int32(_NCHUNK // 2), chunk_pair,
                  jnp.int32(0))
    jt = jnp.int32(_NCHUNK - 1)
    drain(jt, 0)
    compute(jt, 0)

    pltpu.sync_copy(posacc, pos_hbm.at[pl.ds(base, _EW)])
    pltpu.sync_copy(negacc, negv_hbm.at[pl.ds(base, _EW)])


def _tc_reduce_body(pos_ref, neg_ref, out_ref):
    p = pos_ref[...]
    q = neg_ref[...]
    pos_term = -jnp.log(jax.nn.sigmoid(p) + _EPS)
    neg_term = -jnp.log(1.0 - jax.nn.sigmoid(q) + _EPS)
    out_ref[0] = (jnp.sum(pos_term) + jnp.sum(neg_term)) / _E


_tc_reduce = pl.pallas_call(
    _tc_reduce_body,
    out_shape=jax.ShapeDtypeStruct((1,), jnp.float32),
    out_specs=pl.BlockSpec(memory_space=pltpu.SMEM),
)


def kernel(z, edge_index, batch):
    z32 = z.astype(jnp.float32)
    src = edge_index[0].astype(jnp.int32)
    dst = edge_index[1].astype(jnp.int32)
    b32 = batch.astype(jnp.int32)
    counts = jnp.sum(
        b32[None, :] == jnp.arange(_NUM_GRAPHS, dtype=jnp.int32)[:, None],
        axis=1, dtype=jnp.int32)
    starts = (jnp.cumsum(counts) - counts).astype(jnp.int32)
    cnt_tab = jnp.maximum(counts, 1).astype(jnp.int32)
    u = jax.random.uniform(jax.random.key(42), (_E,), dtype=jnp.float32)
    pos_v, neg_v = _sc_dots(z32, src, dst, u, cnt_tab, starts)
    loss = _tc_reduce(pos_v.reshape(_E // _D, _D), neg_v.reshape(_E // _D, _D))
    return loss[0]
